# Initial kernel scaffold; baseline (speedup 1.0000x reference)
import jax
import jax.numpy as jnp
from jax.experimental import pallas as pl


def kernel(x, edge_index, W1, b1, W2, b2):
    raise NotImplementedError("write your pallas kernel here")


# speedup vs baseline: 21.7940x; 21.7940x over previous
import functools

import jax
import jax.numpy as jnp
from jax import lax
from jax.experimental import pallas as pl
from jax.experimental.pallas import tpu as pltpu
from jax.experimental.pallas import tpu_sc as plsc

NC = 2
NS = 16
NW = NC * NS
CHUNK = 128
H = 16


def _sc_degree(col_r, npad):
    niter = col_r.shape[1]
    rpt = npad // NS
    mesh = plsc.VectorSubcoreMesh(core_axis_name="c", subcore_axis_name="s")

    @functools.partial(
        pl.kernel, mesh=mesh,
        compiler_params=pltpu.CompilerParams(use_tc_tiling_on_sc=False),
        out_type=jax.ShapeDtypeStruct((NC, npad), jnp.float32),
        scratch_types=[
            pltpu.VMEM((niter, CHUNK), jnp.int32),
            pltpu.VMEM((CHUNK,), jnp.float32),
            pltpu.VMEM((rpt,), jnp.float32),
            pltpu.VMEM_SHARED((npad,), jnp.float32),
            pltpu.SemaphoreType.DMA,
        ],
    )
    def deg_kernel(col_hbm, deg_hbm, idx_v, ones_v, zeros_v, deg_sh, sem):
        c = lax.axis_index("c")
        s = lax.axis_index("s")
        wid = s * NC + c
        for i in range(CHUNK // 16):
            ones_v[pl.ds(i * 16, 16)] = jnp.ones((16,), jnp.float32)
        for i in range(rpt // 16):
            zeros_v[pl.ds(i * 16, 16)] = jnp.zeros((16,), jnp.float32)
        pltpu.sync_copy(zeros_v, deg_sh.at[pl.ds(s * rpt, rpt)])
        pltpu.async_copy(col_hbm.at[wid], idx_v, sem).wait()
        plsc.subcore_barrier()

        def body(j, carry):
            pltpu.sync_copy(ones_v, deg_sh.at[idx_v.at[j]], add=True)
            return carry

        lax.fori_loop(0, niter, body, 0)
        plsc.subcore_barrier()
        pltpu.sync_copy(deg_sh.at[pl.ds(s * rpt, rpt)],
                        deg_hbm.at[c, pl.ds(s * rpt, rpt)])

    return deg_kernel(col_r)


def _sc_message(g, row_r, col_r):
    npad = g.shape[0]
    niter = row_r.shape[1]
    rpt = npad // NS
    mesh = plsc.VectorSubcoreMesh(core_axis_name="c", subcore_axis_name="s")

    @functools.partial(
        pl.kernel, mesh=mesh,
        compiler_params=pltpu.CompilerParams(use_tc_tiling_on_sc=False),
        out_type=jax.ShapeDtypeStruct((NC, npad, H), jnp.float32),
        scratch_types=[
            pltpu.VMEM((niter, CHUNK), jnp.int32),
            pltpu.VMEM((niter, CHUNK), jnp.int32),
            pltpu.VMEM((CHUNK, H), jnp.float32),
            pltpu.VMEM((16, H), jnp.float32),
            pltpu.VMEM_SHARED((npad, H), jnp.float32),
            pltpu.SemaphoreType.DMA,
        ],
    )
    def msg_kernel(g_hbm, row_hbm, col_hbm, out_hbm,
                   ridx_v, cidx_v, buf, z16, out_sh, sem):
        c = lax.axis_index("c")
        s = lax.axis_index("s")
        wid = s * NC + c
        for i in range(16):
            z16[i, :] = jnp.zeros((H,), jnp.float32)

        def zbody(j, carry):
            pltpu.sync_copy(z16, out_sh.at[pl.ds(s * rpt + j * 16, 16)])
            return carry

        lax.fori_loop(0, rpt // 16, zbody, 0)
        pltpu.async_copy(row_hbm.at[wid], ridx_v, sem).wait()
        pltpu.async_copy(col_hbm.at[wid], cidx_v, sem).wait()
        plsc.subcore_barrier()

        def body(j, carry):
            pltpu.async_copy(g_hbm.at[ridx_v.at[j]], buf, sem).wait()
            pltpu.sync_copy(buf, out_sh.at[cidx_v.at[j]], add=True)
            return carry

        lax.fori_loop(0, niter, body, 0)
        plsc.subcore_barrier()
        pltpu.sync_copy(out_sh.at[pl.ds(s * rpt, rpt)],
                        out_hbm.at[c, pl.ds(s * rpt, rpt)])

    return msg_kernel(g, row_r, col_r)


def _dinv_of(deg_ref):
    deg = deg_ref[:, 0] + deg_ref[:, 1] + 1.0
    return lax.rsqrt(deg)


def _tc_linear1(x, w1, b1, degp_t, bm):
    n, d = x.shape
    h = w1.shape[0]

    def body(x_ref, w_ref, b_ref, deg_ref, g_ref):
        hid = lax.dot_general(x_ref[...], w_ref[...],
                              (((1,), (1,)), ((), ())),
                              preferred_element_type=jnp.float32)
        hid = hid + b_ref[...][None, :]
        g_ref[...] = _dinv_of(deg_ref)[:, None] * hid

    return pl.pallas_call(
        body,
        grid=(n // bm,),
        in_specs=[
            pl.BlockSpec((bm, d), lambda i: (i, 0)),
            pl.BlockSpec((h, d), lambda i: (0, 0)),
            pl.BlockSpec((h,), lambda i: (0,)),
            pl.BlockSpec((bm, NC), lambda i: (i, 0)),
        ],
        out_specs=pl.BlockSpec((bm, h), lambda i: (i, 0)),
        out_shape=jax.ShapeDtypeStruct((n, h), jnp.float32),
    )(x, w1, b1, degp_t)


def _tc_mid(s1, g1p, degp_t, w2p, b2p, n, bm):
    npad = g1p.shape[0]

    def body(s_ref, g_ref, deg_ref, w_ref, b_ref, o_ref):
        dinv = _dinv_of(deg_ref)
        acc = s_ref[0] + s_ref[1] + g_ref[...]
        h1 = jnp.maximum(dinv[:, None] * acc, 0.0)
        h2 = lax.dot_general(h1, w_ref[...], (((1,), (1,)), ((), ())),
                             preferred_element_type=jnp.float32)
        h2 = h2 + b_ref[...][None, :]
        o_ref[...] = dinv[:, None] * h2

    return pl.pallas_call(
        body,
        grid=(n // bm,),
        in_specs=[
            pl.BlockSpec((NC, bm, H), lambda i: (0, i, 0)),
            pl.BlockSpec((bm, H), lambda i: (i, 0)),
            pl.BlockSpec((bm, NC), lambda i: (i, 0)),
            pl.BlockSpec((H, H), lambda i: (0, 0)),
            pl.BlockSpec((H,), lambda i: (0,)),
        ],
        out_specs=pl.BlockSpec((bm, H), lambda i: (i, 0)),
        out_shape=jax.ShapeDtypeStruct((n, H), jnp.float32),
    )(s1, g1p, degp_t, w2p, b2p)


def _tc_out(s2, g2p, degp_t, n, ncls, bm):
    def body(s_ref, g_ref, deg_ref, o_ref):
        dinv = _dinv_of(deg_ref)
        o = dinv[:, None] * (s_ref[0] + s_ref[1] + g_ref[...])
        logits = o[:, :ncls]
        m = jnp.max(logits, axis=1, keepdims=True)
        lse = jnp.log(jnp.sum(jnp.exp(logits - m), axis=1, keepdims=True)) + m
        o_ref[...] = logits - lse

    return pl.pallas_call(
        body,
        grid=(n // bm,),
        in_specs=[
            pl.BlockSpec((NC, bm, H), lambda i: (0, i, 0)),
            pl.BlockSpec((bm, H), lambda i: (i, 0)),
            pl.BlockSpec((bm, NC), lambda i: (i, 0)),
        ],
        out_specs=pl.BlockSpec((bm, ncls), lambda i: (i, 0)),
        out_shape=jax.ShapeDtypeStruct((n, ncls), jnp.float32),
    )(s2, g2p, degp_t)


def kernel(x, edge_index, W1, b1, W2, b2):
    n, d = x.shape
    hid = W1.shape[0]
    ncls = W2.shape[0]
    assert hid == H

    npad = ((n + NW * 16 - 1) // (NW * 16)) * NW * 16
    padn = n
    e = edge_index.shape[1]
    epad = ((e + NW * CHUNK - 1) // (NW * CHUNK)) * NW * CHUNK
    niter = epad // (NW * CHUNK)

    ei = edge_index.astype(jnp.int32)
    pad = jnp.full((epad - e,), padn, jnp.int32)
    row_r = jnp.concatenate([ei[0], pad]).reshape(NW, niter, CHUNK)
    col_r = jnp.concatenate([ei[1], pad]).reshape(NW, niter, CHUNK)

    w2p = jnp.zeros((H, H), jnp.float32).at[:ncls].set(W2)
    b2p = jnp.zeros((H,), jnp.float32).at[:ncls].set(b2)

    bm = 1000 if n % 1000 == 0 else 8

    degp = _sc_degree(col_r, npad)
    degp_t = degp.T

    g1 = _tc_linear1(x, W1, b1, degp_t, bm)
    g1p = jnp.concatenate([g1, jnp.zeros((npad - n, H), jnp.float32)])
    s1 = _sc_message(g1p, row_r, col_r)

    g2 = _tc_mid(s1, g1p, degp_t, w2p, b2p, n, bm)
    g2p = jnp.concatenate([g2, jnp.zeros((npad - n, H), jnp.float32)])
    s2 = _sc_message(g2p, row_r, col_r)

    return _tc_out(s2, g2p, degp_t, n, ncls, bm)


# speedup vs baseline: 21.9854x; 1.0088x over previous
import functools

import jax
import jax.numpy as jnp
from jax import lax
from jax.experimental import pallas as pl
from jax.experimental.pallas import tpu as pltpu
from jax.experimental.pallas import tpu_sc as plsc

NC = 2
NS = 16
NW = NC * NS
CHUNK = 128
H = 16


def _sc_degree(col_r, npad):
    niter = col_r.shape[1]
    rpt = npad // NS
    mesh = plsc.VectorSubcoreMesh(core_axis_name="c", subcore_axis_name="s")

    @functools.partial(
        pl.kernel, mesh=mesh,
        compiler_params=pltpu.CompilerParams(use_tc_tiling_on_sc=False),
        out_type=jax.ShapeDtypeStruct((NC, npad), jnp.float32),
        scratch_types=[
            pltpu.VMEM((niter, CHUNK), jnp.int32),
            pltpu.VMEM((CHUNK,), jnp.float32),
            pltpu.VMEM((rpt,), jnp.float32),
            pltpu.VMEM_SHARED((npad,), jnp.float32),
            pltpu.SemaphoreType.DMA,
        ],
    )
    def deg_kernel(col_hbm, deg_hbm, idx_v, ones_v, zeros_v, deg_sh, sem):
        c = lax.axis_index("c")
        s = lax.axis_index("s")
        wid = s * NC + c
        for i in range(CHUNK // 16):
            ones_v[pl.ds(i * 16, 16)] = jnp.ones((16,), jnp.float32)
        for i in range(rpt // 16):
            zeros_v[pl.ds(i * 16, 16)] = jnp.zeros((16,), jnp.float32)
        pltpu.sync_copy(zeros_v, deg_sh.at[pl.ds(s * rpt, rpt)])
        pltpu.async_copy(col_hbm.at[wid], idx_v, sem).wait()
        plsc.subcore_barrier()

        def body(j, carry):
            pltpu.sync_copy(ones_v, deg_sh.at[idx_v.at[j]], add=True)
            return carry

        lax.fori_loop(0, niter, body, 0)
        plsc.subcore_barrier()
        pltpu.sync_copy(deg_sh.at[pl.ds(s * rpt, rpt)],
                        deg_hbm.at[c, pl.ds(s * rpt, rpt)])

    return deg_kernel(col_r)


def _sc_message(g, row_r, col_r, npad):
    niter = row_r.shape[1]
    assert niter % 2 == 0
    rpt = npad // NS
    mesh = plsc.VectorSubcoreMesh(core_axis_name="c", subcore_axis_name="s")

    @functools.partial(
        pl.kernel, mesh=mesh,
        compiler_params=pltpu.CompilerParams(use_tc_tiling_on_sc=False),
        out_type=jax.ShapeDtypeStruct((NC, npad, H), jnp.float32),
        scratch_types=[
            pltpu.VMEM((niter + 1, CHUNK), jnp.int32),
            pltpu.VMEM((niter, CHUNK), jnp.int32),
            pltpu.VMEM((CHUNK, H), jnp.float32),
            pltpu.VMEM((CHUNK, H), jnp.float32),
            pltpu.VMEM((16, H), jnp.float32),
            pltpu.VMEM_SHARED((npad, H), jnp.float32),
            pltpu.SemaphoreType.DMA,
            pltpu.SemaphoreType.DMA,
        ],
    )
    def msg_kernel(g_hbm, row_hbm, col_hbm, out_hbm,
                   ridx_v, cidx_v, buf0, buf1, z16, out_sh, sem0, sem1):
        c = lax.axis_index("c")
        s = lax.axis_index("s")
        wid = s * NC + c
        for i in range(16):
            z16[i, :] = jnp.zeros((H,), jnp.float32)
        for i in range(CHUNK // 16):
            ridx_v[niter, pl.ds(i * 16, 16)] = jnp.zeros((16,), jnp.int32)

        def zbody(j, carry):
            pltpu.sync_copy(z16, out_sh.at[pl.ds(s * rpt + j * 16, 16)])
            return carry

        lax.fori_loop(0, rpt // 16, zbody, 0)
        pltpu.async_copy(row_hbm.at[wid], ridx_v.at[pl.ds(0, niter)],
                         sem0).wait()
        pltpu.async_copy(col_hbm.at[wid], cidx_v, sem1).wait()
        plsc.subcore_barrier()

        pltpu.async_copy(g_hbm.at[ridx_v.at[0]], buf0, sem0)

        def body(j2, carry):
            j = j2 * 2
            pltpu.async_copy(g_hbm.at[ridx_v.at[j + 1]], buf1, sem1)
            pltpu.make_async_copy(g_hbm.at[ridx_v.at[j]], buf0, sem0).wait()
            pltpu.sync_copy(buf0, out_sh.at[cidx_v.at[j]], add=True)
            pltpu.async_copy(g_hbm.at[ridx_v.at[j + 2]], buf0, sem0)
            pltpu.make_async_copy(g_hbm.at[ridx_v.at[j]], buf1, sem1).wait()
            pltpu.sync_copy(buf1, out_sh.at[cidx_v.at[j + 1]], add=True)
            return carry

        lax.fori_loop(0, niter // 2, body, 0)
        pltpu.make_async_copy(g_hbm.at[ridx_v.at[0]], buf0, sem0).wait()
        plsc.subcore_barrier()
        pltpu.sync_copy(out_sh.at[pl.ds(s * rpt, rpt)],
                        out_hbm.at[c, pl.ds(s * rpt, rpt)])

    return msg_kernel(g, row_r, col_r)


def _dinv_of(deg_ref):
    deg = deg_ref[:, 0] + deg_ref[:, 1] + 1.0
    return lax.rsqrt(deg)


def _tc_linear1(x, w1, b1, degp_t, bm):
    n, d = x.shape
    h = w1.shape[0]

    def body(x_ref, w_ref, b_ref, deg_ref, g_ref):
        hid = lax.dot_general(x_ref[...], w_ref[...],
                              (((1,), (1,)), ((), ())),
                              preferred_element_type=jnp.float32)
        hid = hid + b_ref[...][None, :]
        g_ref[...] = _dinv_of(deg_ref)[:, None] * hid

    return pl.pallas_call(
        body,
        grid=(n // bm,),
        in_specs=[
            pl.BlockSpec((bm, d), lambda i: (i, 0)),
            pl.BlockSpec((h, d), lambda i: (0, 0)),
            pl.BlockSpec((h,), lambda i: (0,)),
            pl.BlockSpec((bm, NC), lambda i: (i, 0)),
        ],
        out_specs=pl.BlockSpec((bm, h), lambda i: (i, 0)),
        out_shape=jax.ShapeDtypeStruct((n, h), jnp.float32),
    )(x, w1, b1, degp_t)


def _tc_mid(s1, g1p, degp_t, w2p, b2p, n, bm):
    npad = g1p.shape[0]

    def body(s_ref, g_ref, deg_ref, w_ref, b_ref, o_ref):
        dinv = _dinv_of(deg_ref)
        acc = s_ref[0] + s_ref[1] + g_ref[...]
        h1 = jnp.maximum(dinv[:, None] * acc, 0.0)
        h2 = lax.dot_general(h1, w_ref[...], (((1,), (1,)), ((), ())),
                             preferred_element_type=jnp.float32)
        h2 = h2 + b_ref[...][None, :]
        o_ref[...] = dinv[:, None] * h2

    return pl.pallas_call(
        body,
        grid=(n // bm,),
        in_specs=[
            pl.BlockSpec((NC, bm, H), lambda i: (0, i, 0)),
            pl.BlockSpec((bm, H), lambda i: (i, 0)),
            pl.BlockSpec((bm, NC), lambda i: (i, 0)),
            pl.BlockSpec((H, H), lambda i: (0, 0)),
            pl.BlockSpec((H,), lambda i: (0,)),
        ],
        out_specs=pl.BlockSpec((bm, H), lambda i: (i, 0)),
        out_shape=jax.ShapeDtypeStruct((n, H), jnp.float32),
    )(s1, g1p, degp_t, w2p, b2p)


def _tc_out(s2, g2p, degp_t, n, ncls, bm):
    def body(s_ref, g_ref, deg_ref, o_ref):
        dinv = _dinv_of(deg_ref)
        o = dinv[:, None] * (s_ref[0] + s_ref[1] + g_ref[...])
        logits = o[:, :ncls]
        m = jnp.max(logits, axis=1, keepdims=True)
        lse = jnp.log(jnp.sum(jnp.exp(logits - m), axis=1, keepdims=True)) + m
        o_ref[...] = logits - lse

    return pl.pallas_call(
        body,
        grid=(n // bm,),
        in_specs=[
            pl.BlockSpec((NC, bm, H), lambda i: (0, i, 0)),
            pl.BlockSpec((bm, H), lambda i: (i, 0)),
            pl.BlockSpec((bm, NC), lambda i: (i, 0)),
        ],
        out_specs=pl.BlockSpec((bm, ncls), lambda i: (i, 0)),
        out_shape=jax.ShapeDtypeStruct((n, ncls), jnp.float32),
    )(s2, g2p, degp_t)


def kernel(x, edge_index, W1, b1, W2, b2):
    n, d = x.shape
    hid = W1.shape[0]
    ncls = W2.shape[0]
    assert hid == H

    npad = ((n + NW * 16 - 1) // (NW * 16)) * NW * 16
    e = edge_index.shape[1]
    epad = ((e + 2 * NW * CHUNK - 1) // (2 * NW * CHUNK)) * 2 * NW * CHUNK
    niter = epad // (NW * CHUNK)

    ei = edge_index.astype(jnp.int32)
    row_r = jnp.concatenate(
        [ei[0], jnp.zeros((epad - e,), jnp.int32)]).reshape(NW, niter, CHUNK)
    col_r = jnp.concatenate(
        [ei[1], jnp.full((epad - e,), n, jnp.int32)]).reshape(NW, niter, CHUNK)

    w2p = jnp.zeros((H, H), jnp.float32).at[:ncls].set(W2)
    b2p = jnp.zeros((H,), jnp.float32).at[:ncls].set(b2)

    bm = 1000 if n % 1000 == 0 else 8

    degp = _sc_degree(col_r, npad)
    degp_t = degp.T

    g1 = _tc_linear1(x, W1, b1, degp_t, bm)
    s1 = _sc_message(g1, row_r, col_r, npad)

    g2 = _tc_mid(s1, g1, degp_t, w2p, b2p, n, bm)
    s2 = _sc_message(g2, row_r, col_r, npad)

    return _tc_out(s2, g2, degp_t, n, ncls, bm)


# speedup vs baseline: 26.1859x; 1.1911x over previous
import functools

import jax
import jax.numpy as jnp
from jax import lax
from jax.experimental import pallas as pl
from jax.experimental.pallas import tpu as pltpu
from jax.experimental.pallas import tpu_sc as plsc

NC = 2
NS = 16
NW = NC * NS
NQ = 4
NP = 4
K = 2048
H = 16
BM = 1024


def _sc_degree(col32, npad):
    epw = col32.shape[2]
    mesh = plsc.VectorSubcoreMesh(core_axis_name="c", subcore_axis_name="s")

    @functools.partial(
        pl.kernel, mesh=mesh,
        compiler_params=pltpu.CompilerParams(use_tc_tiling_on_sc=False, needs_layout_passes=False),
        out_type=jax.ShapeDtypeStruct((NC, NS, npad), jnp.float32),
        scratch_types=[
            pltpu.VMEM((npad,), jnp.float32),
            pltpu.VMEM((epw,), jnp.int32),
            pltpu.SemaphoreType.DMA,
        ],
    )
    def deg_kernel(col_hbm, deg_hbm, acc_v, idx_v, sem):
        c = lax.axis_index("c")
        s = lax.axis_index("s")
        pltpu.async_copy(col_hbm.at[c, s], idx_v, sem)

        def zbody(j, carry):
            for u in range(8):
                acc_v[pl.ds((j * 8 + u) * 16, 16)] = jnp.zeros((16,),
                                                               jnp.float32)
            return carry

        lax.fori_loop(0, npad // 128, zbody, 0)
        pltpu.make_async_copy(col_hbm.at[c, s], idx_v, sem).wait()
        ones = jnp.ones((16,), jnp.float32)

        def body(i, carry):
            cc = idx_v[pl.ds(i * 16, 16)]
            plsc.addupdate_scatter(acc_v, [cc], ones)
            return carry

        lax.fori_loop(0, epw // 16, body, 0)
        pltpu.sync_copy(acc_v, deg_hbm.at[c, s])

    return deg_kernel(col32)


def _sc_message(gt, row8, col8, npad):
    nchunk = row8.shape[2] - 1
    assert nchunk % 2 == 0
    hq = H // NQ
    mesh = plsc.VectorSubcoreMesh(core_axis_name="c", subcore_axis_name="s")

    @functools.partial(
        pl.kernel, mesh=mesh,
        compiler_params=pltpu.CompilerParams(use_tc_tiling_on_sc=False, needs_layout_passes=False),
        out_type=jax.ShapeDtypeStruct((NC, NS, NQ, npad), jnp.float32),
        scratch_types=[
            pltpu.VMEM((hq, npad), jnp.float32),
            pltpu.VMEM((hq, npad), jnp.float32),
            pltpu.VMEM((2, K), jnp.int32),
            pltpu.VMEM((2, K), jnp.int32),
            pltpu.SemaphoreType.DMA,
            pltpu.SemaphoreType.DMA,
            pltpu.SemaphoreType.DMA,
        ],
    )
    def msg_kernel(gt_hbm, row_hbm, col_hbm, out_hbm,
                   gq_v, acc_v, ridx, cidx, semg, sem0, sem1):
        c = lax.axis_index("c")
        s = lax.axis_index("s")
        q = s % NQ
        p = s // NQ
        pltpu.async_copy(gt_hbm.at[pl.ds(q * hq, hq)], gq_v, semg)
        pltpu.async_copy(row_hbm.at[c, p, 0], ridx.at[0], sem0)
        pltpu.async_copy(col_hbm.at[c, p, 0], cidx.at[0], sem0)

        def zbody(j, carry):
            for f in range(hq):
                for u in range(2):
                    acc_v[f, pl.ds((j * 2 + u) * 16, 16)] = jnp.zeros(
                        (16,), jnp.float32)
            return carry

        lax.fori_loop(0, npad // 32, zbody, 0)
        pltpu.make_async_copy(gt_hbm.at[pl.ds(0, hq)], gq_v, semg).wait()

        fvecs = [jnp.full((16,), f, jnp.int32) for f in range(hq)]

        def chunk(t, buf, sem_cur, sem_nxt, rb, cb):
            nxt = 1 - buf
            pltpu.async_copy(row_hbm.at[c, p, t + 1], ridx.at[nxt], sem_nxt)
            pltpu.async_copy(col_hbm.at[c, p, t + 1], cidx.at[nxt], sem_nxt)
            pltpu.make_async_copy(row_hbm.at[c, p, 0], rb, sem_cur).wait()
            pltpu.make_async_copy(col_hbm.at[c, p, 0], cb, sem_cur).wait()

            def ibody(i, carry):
                r = rb[pl.ds(i * 16, 16)]
                cc = cb[pl.ds(i * 16, 16)]
                for f in range(hq):
                    v = plsc.load_gather(gq_v, [fvecs[f], r])
                    plsc.addupdate_scatter(acc_v, [fvecs[f], cc], v)
                return carry

            lax.fori_loop(0, K // 16, ibody, 0)

        def body(t2, carry):
            t = t2 * 2
            chunk(t, 0, sem0, sem1, ridx.at[0], cidx.at[0])
            chunk(t + 1, 1, sem1, sem0, ridx.at[1], cidx.at[1])
            return carry

        lax.fori_loop(0, nchunk // 2, body, 0)
        pltpu.make_async_copy(row_hbm.at[c, p, 0], ridx.at[0], sem0).wait()
        pltpu.make_async_copy(col_hbm.at[c, p, 0], cidx.at[0], sem0).wait()
        pltpu.sync_copy(acc_v, out_hbm.at[c, s])

    return msg_kernel(gt, row8, col8)


def _tc_linear1(x, w1, b1, degp, npad):
    n, d = x.shape

    def body(x_ref, w_ref, b_ref, deg_ref, g_ref, dinv_ref):
        deg = jnp.sum(deg_ref[...], axis=(0, 1)) + 1.0
        dinv = lax.rsqrt(deg)
        hid = lax.dot_general(w_ref[...], x_ref[...],
                              (((1,), (1,)), ((), ())),
                              preferred_element_type=jnp.float32)
        g_ref[...] = dinv[None, :] * (hid + b_ref[...][:, None])
        dinv_ref[...] = dinv

    return pl.pallas_call(
        body,
        grid=(npad // BM,),
        in_specs=[
            pl.BlockSpec((BM, d), lambda i: (i, 0)),
            pl.BlockSpec((H, d), lambda i: (0, 0)),
            pl.BlockSpec((H,), lambda i: (0,)),
            pl.BlockSpec((NC, NS, BM), lambda i: (0, 0, i)),
        ],
        out_specs=[
            pl.BlockSpec((H, BM), lambda i: (0, i)),
            pl.BlockSpec((BM,), lambda i: (i,)),
        ],
        out_shape=[
            jax.ShapeDtypeStruct((H, npad), jnp.float32),
            jax.ShapeDtypeStruct((npad,), jnp.float32),
        ],
    )(x, w1, b1, degp)


def _assemble(s_ref):
    rows = []
    for q in range(NQ):
        t = None
        for c in range(NC):
            for p in range(NP):
                term = s_ref[c, p * NQ + q]
                t = term if t is None else t + term
        rows.append(t)
    return jnp.concatenate(rows, axis=0)


def _tc_mid(s1, gt, dinv, w2p, b2p, npad):
    def body(s_ref, g_ref, dinv_ref, w_ref, b_ref, o_ref):
        di = dinv_ref[...]
        h1 = jnp.maximum(di[None, :] * (_assemble(s_ref) + g_ref[...]), 0.0)
        h2 = lax.dot_general(w_ref[...], h1, (((1,), (0,)), ((), ())),
                             preferred_element_type=jnp.float32)
        o_ref[...] = di[None, :] * (h2 + b_ref[...][:, None])

    return pl.pallas_call(
        body,
        grid=(npad // BM,),
        in_specs=[
            pl.BlockSpec((NC, NS, NQ, BM), lambda i: (0, 0, 0, i)),
            pl.BlockSpec((H, BM), lambda i: (0, i)),
            pl.BlockSpec((BM,), lambda i: (i,)),
            pl.BlockSpec((H, H), lambda i: (0, 0)),
            pl.BlockSpec((H,), lambda i: (0,)),
        ],
        out_specs=pl.BlockSpec((H, BM), lambda i: (0, i)),
        out_shape=jax.ShapeDtypeStruct((H, npad), jnp.float32),
    )(s1, gt, dinv, w2p, b2p)


def _tc_out(s2, g2t, dinv, ncls, npad):
    def body(s_ref, g_ref, dinv_ref, o_ref):
        di = dinv_ref[...]
        o = di[None, :] * (_assemble(s_ref) + g_ref[...])
        logits = o[:ncls, :]
        m = jnp.max(logits, axis=0, keepdims=True)
        lse = jnp.log(jnp.sum(jnp.exp(logits - m), axis=0, keepdims=True))
        o_ref[...] = logits - m - lse

    return pl.pallas_call(
        body,
        grid=(npad // BM,),
        in_specs=[
            pl.BlockSpec((NC, NS, NQ, BM), lambda i: (0, 0, 0, i)),
            pl.BlockSpec((H, BM), lambda i: (0, i)),
            pl.BlockSpec((BM,), lambda i: (i,)),
        ],
        out_specs=pl.BlockSpec((ncls, BM), lambda i: (0, i)),
        out_shape=jax.ShapeDtypeStruct((ncls, npad), jnp.float32),
    )(s2, g2t, dinv)


def kernel(x, edge_index, W1, b1, W2, b2):
    n, d = x.shape
    hid = W1.shape[0]
    ncls = W2.shape[0]
    assert hid == H

    npad = ((n + 2 * BM - 1) // (2 * BM)) * 2 * BM
    e = edge_index.shape[1]
    nparts = NC * NP
    epp = 2 * K
    epad = ((e + nparts * epp - 1) // (nparts * epp)) * nparts * epp
    nchunk = epad // (nparts * K)

    ei = edge_index.astype(jnp.int32)
    rowp = jnp.concatenate([ei[0], jnp.zeros((epad - e,), jnp.int32)])
    colp = jnp.concatenate([ei[1], jnp.full((epad - e,), n, jnp.int32)])
    row8 = jnp.concatenate(
        [rowp.reshape(nparts, nchunk, K),
         jnp.zeros((nparts, 1, K), jnp.int32)], axis=1).reshape(
             NC, NP, nchunk + 1, K)
    col8 = jnp.concatenate(
        [colp.reshape(nparts, nchunk, K),
         jnp.full((nparts, 1, K), n, jnp.int32)], axis=1).reshape(
             NC, NP, nchunk + 1, K)
    col32 = colp.reshape(NC, NS, epad // NW)

    w2p = jnp.zeros((H, H), jnp.float32).at[:ncls].set(W2)
    b2p = jnp.zeros((H,), jnp.float32).at[:ncls].set(b2)

    degp = _sc_degree(col32, npad)
    gt, dinv = _tc_linear1(x, W1, b1, degp, npad)
    s1 = _sc_message(gt, row8, col8, npad)
    g2t = _tc_mid(s1, gt, dinv, w2p, b2p, npad)
    s2 = _sc_message(g2t, row8, col8, npad)
    out_t = _tc_out(s2, g2t, dinv, ncls, npad)
    return out_t[:, :n].T


# speedup vs baseline: 34.0230x; 1.2993x over previous
import functools

import jax
import jax.numpy as jnp
from jax import lax
from jax.experimental import pallas as pl
from jax.experimental.pallas import tpu as pltpu
from jax.experimental.pallas import tpu_sc as plsc

NC = 2
NS = 16
NW = NC * NS
NQ = 4
NP = 4
K = 2048
H = 16
BM = 1024


def _sc_degree(eip, npad, epw):
    mesh = plsc.VectorSubcoreMesh(core_axis_name="c", subcore_axis_name="s")

    @functools.partial(
        pl.kernel, mesh=mesh,
        compiler_params=pltpu.CompilerParams(use_tc_tiling_on_sc=False,
                                             needs_layout_passes=False),
        out_type=jax.ShapeDtypeStruct((NC, NS, npad), jnp.float32),
        scratch_types=[
            pltpu.VMEM((npad,), jnp.float32),
            pltpu.VMEM((epw,), jnp.int32),
            pltpu.SemaphoreType.DMA,
        ],
    )
    def deg_kernel(e_hbm, deg_hbm, acc_v, idx_v, sem):
        c = lax.axis_index("c")
        s = lax.axis_index("s")
        w = c * NS + s
        pltpu.async_copy(e_hbm.at[1, pl.ds(w * epw, epw)], idx_v, sem)

        @plsc.parallel_loop(0, npad // 16, unroll=8)
        def _z(j):
            acc_v[pl.ds(j * 16, 16)] = jnp.zeros((16,), jnp.float32)

        pltpu.make_async_copy(e_hbm.at[1, pl.ds(0, epw)], idx_v, sem).wait()
        ones = jnp.ones((16,), jnp.float32)

        @plsc.parallel_loop(0, epw // 16, unroll=4)
        def _h(i):
            cc = idx_v[pl.ds(i * 16, 16)]
            plsc.addupdate_scatter(acc_v, [cc], ones)

        pltpu.sync_copy(acc_v, deg_hbm.at[c, s])

    return deg_kernel(eip)


def _sc_message(gt, eip, npad, nch0, nch1, k2):
    assert nch0 % 2 == 0 and nch1 % 2 == 0
    hq = H // NQ
    mesh = plsc.VectorSubcoreMesh(core_axis_name="c", subcore_axis_name="s")

    @functools.partial(
        pl.kernel, mesh=mesh,
        compiler_params=pltpu.CompilerParams(use_tc_tiling_on_sc=False,
                                             needs_layout_passes=False),
        out_type=jax.ShapeDtypeStruct((NC, NS, NQ, npad), jnp.float32),
        scratch_types=[
            pltpu.VMEM((hq, npad), jnp.float32),
            pltpu.VMEM((hq, npad), jnp.float32),
            pltpu.VMEM((2, k2), jnp.int32),
            pltpu.VMEM((2, k2), jnp.int32),
            pltpu.SemaphoreType.DMA,
            pltpu.SemaphoreType.DMA,
            pltpu.SemaphoreType.DMA,
        ],
    )
    def msg_kernel(gt_hbm, e_hbm, out_hbm,
                   gq_v, acc_v, ridx, cidx, semg, sem0, sem1):
        c = lax.axis_index("c")
        s = lax.axis_index("s")
        q = s % NQ
        p = s // NQ
        nch = lax.select(c == 0, nch0, nch1)
        base = lax.select(c == 0, p * nch0, NP * nch0 + p * nch1) * k2

        def ld(t, buf, sem):
            off = base + t * k2
            pltpu.async_copy(e_hbm.at[0, pl.ds(off, k2)], ridx.at[buf], sem)
            pltpu.async_copy(e_hbm.at[1, pl.ds(off, k2)], cidx.at[buf], sem)

        pltpu.async_copy(gt_hbm.at[pl.ds(q * hq, hq)], gq_v, semg)
        ld(0, 0, sem0)

        @plsc.parallel_loop(0, npad // 16, unroll=8)
        def _z(j):
            for f in range(hq):
                acc_v[f, pl.ds(j * 16, 16)] = jnp.zeros((16,), jnp.float32)

        pltpu.make_async_copy(gt_hbm.at[pl.ds(0, hq)], gq_v, semg).wait()

        fvecs = [jnp.full((16,), f, jnp.int32) for f in range(hq)]

        def chunk(t, buf, sem_cur, sem_nxt):
            @pl.when(t + 1 < nch)
            def _():
                ld(t + 1, 1 - buf, sem_nxt)

            rb = ridx.at[buf]
            cb = cidx.at[buf]
            pltpu.make_async_copy(e_hbm.at[0, pl.ds(0, k2)], rb,
                                  sem_cur).wait()
            pltpu.make_async_copy(e_hbm.at[0, pl.ds(0, k2)], cb,
                                  sem_cur).wait()

            @plsc.parallel_loop(0, k2 // 16, unroll=4)
            def _i(i):
                r = rb[pl.ds(i * 16, 16)]
                cc = cb[pl.ds(i * 16, 16)]
                for f in range(hq):
                    v = plsc.load_gather(gq_v, [fvecs[f], r])
                    plsc.addupdate_scatter(acc_v, [fvecs[f], cc], v)

        def body(t2, carry):
            t = t2 * 2
            chunk(t, 0, sem0, sem1)
            chunk(t + 1, 1, sem1, sem0)
            return carry

        lax.fori_loop(0, nch // 2, body, 0)
        pltpu.sync_copy(acc_v, out_hbm.at[c, s])

    return msg_kernel(gt, eip)


def _tc_linear1(x, w1, b1, degp, npad):
    n, d = x.shape

    def body(x_ref, w_ref, b_ref, deg_ref, g_ref, dinv_ref):
        deg = jnp.sum(deg_ref[...], axis=(0, 1)) + 1.0
        dinv = lax.rsqrt(deg)
        hid = lax.dot_general(w_ref[...], x_ref[...],
                              (((1,), (1,)), ((), ())),
                              preferred_element_type=jnp.float32)
        g_ref[...] = dinv[None, :] * (hid + b_ref[...][:, None])
        dinv_ref[...] = dinv

    return pl.pallas_call(
        body,
        grid=(npad // BM,),
        in_specs=[
            pl.BlockSpec((BM, d), lambda i: (i, 0)),
            pl.BlockSpec((H, d), lambda i: (0, 0)),
            pl.BlockSpec((H,), lambda i: (0,)),
            pl.BlockSpec((NC, NS, BM), lambda i: (0, 0, i)),
        ],
        out_specs=[
            pl.BlockSpec((H, BM), lambda i: (0, i)),
            pl.BlockSpec((BM,), lambda i: (i,)),
        ],
        out_shape=[
            jax.ShapeDtypeStruct((H, npad), jnp.float32),
            jax.ShapeDtypeStruct((npad,), jnp.float32),
        ],
    )(x, w1, b1, degp)


def _assemble(s_ref):
    rows = []
    for q in range(NQ):
        t = None
        for c in range(NC):
            for p in range(NP):
                term = s_ref[c, p * NQ + q]
                t = term if t is None else t + term
        rows.append(t)
    return jnp.concatenate(rows, axis=0)


def _tc_mid(s1, gt, dinv, w2p, b2p, npad):
    def body(s_ref, g_ref, dinv_ref, w_ref, b_ref, o_ref):
        di = dinv_ref[...]
        h1 = jnp.maximum(di[None, :] * (_assemble(s_ref) + g_ref[...]), 0.0)
        h2 = lax.dot_general(w_ref[...], h1, (((1,), (0,)), ((), ())),
                             preferred_element_type=jnp.float32)
        o_ref[...] = di[None, :] * (h2 + b_ref[...][:, None])

    return pl.pallas_call(
        body,
        grid=(npad // BM,),
        in_specs=[
            pl.BlockSpec((NC, NS, NQ, BM), lambda i: (0, 0, 0, i)),
            pl.BlockSpec((H, BM), lambda i: (0, i)),
            pl.BlockSpec((BM,), lambda i: (i,)),
            pl.BlockSpec((H, H), lambda i: (0, 0)),
            pl.BlockSpec((H,), lambda i: (0,)),
        ],
        out_specs=pl.BlockSpec((H, BM), lambda i: (0, i)),
        out_shape=jax.ShapeDtypeStruct((H, npad), jnp.float32),
    )(s1, gt, dinv, w2p, b2p)


def _tc_out(s2, g2t, dinv, ncls, npad):
    def body(s_ref, g_ref, dinv_ref, o_ref):
        di = dinv_ref[...]
        o = di[None, :] * (_assemble(s_ref) + g_ref[...])
        logits = o[:ncls, :]
        m = jnp.max(logits, axis=0, keepdims=True)
        lse = jnp.log(jnp.sum(jnp.exp(logits - m), axis=0, keepdims=True))
        o_ref[...] = logits - m - lse

    return pl.pallas_call(
        body,
        grid=(npad // BM,),
        in_specs=[
            pl.BlockSpec((NC, NS, NQ, BM), lambda i: (0, 0, 0, i)),
            pl.BlockSpec((H, BM), lambda i: (0, i)),
            pl.BlockSpec((BM,), lambda i: (i,)),
        ],
        out_specs=pl.BlockSpec((ncls, BM), lambda i: (0, i)),
        out_shape=jax.ShapeDtypeStruct((ncls, npad), jnp.float32),
    )(s2, g2t, dinv)


def kernel(x, edge_index, W1, b1, W2, b2):
    n, d = x.shape
    hid = W1.shape[0]
    ncls = W2.shape[0]
    assert hid == H

    npad = ((n + 2 * BM - 1) // (2 * BM)) * 2 * BM
    e = edge_index.shape[1]
    k2 = 1024
    unit = 2 * NC * NP * k2
    epad = ((e + unit - 1) // unit) * unit
    pairs = epad // (NP * 2 * k2)
    a2 = max(1, min(pairs - 1, round(0.55 * pairs)))
    nch0, nch1 = 2 * a2, 2 * (pairs - a2)
    epw = epad // NW

    ei = edge_index.astype(jnp.int32)
    eip = jnp.concatenate(
        [ei, jnp.stack([jnp.zeros((epad - e,), jnp.int32),
                        jnp.full((epad - e,), n, jnp.int32)])], axis=1)

    w2p = jnp.zeros((H, H), jnp.float32).at[:ncls].set(W2)
    b2p = jnp.zeros((H,), jnp.float32).at[:ncls].set(b2)

    degp = _sc_degree(eip, npad, epw)
    gt, dinv = _tc_linear1(x, W1, b1, degp, npad)
    s1 = _sc_message(gt, eip, npad, nch0, nch1, k2)
    g2t = _tc_mid(s1, gt, dinv, w2p, b2p, npad)
    s2 = _sc_message(g2t, eip, npad, nch0, nch1, k2)
    out_t = _tc_out(s2, g2t, dinv, ncls, npad)
    return out_t[:, :n].T


# speedup vs baseline: 35.2976x; 1.0375x over previous
import functools

import jax
import jax.numpy as jnp
from jax import lax
from jax.experimental import pallas as pl
from jax.experimental.pallas import tpu as pltpu
from jax.experimental.pallas import tpu_sc as plsc

NC = 2
NS = 16
NW = NC * NS
NQ = 4
NP = 4
K = 2048
H = 16
BM = 1024


def _sc_degree(eip, npad, epw):
    mesh = plsc.VectorSubcoreMesh(core_axis_name="c", subcore_axis_name="s")

    @functools.partial(
        pl.kernel, mesh=mesh,
        compiler_params=pltpu.CompilerParams(use_tc_tiling_on_sc=False,
                                             needs_layout_passes=False),
        out_type=jax.ShapeDtypeStruct((NC, NS, npad), jnp.float32),
        scratch_types=[
            pltpu.VMEM((npad,), jnp.float32),
            pltpu.VMEM((epw,), jnp.int32),
            pltpu.SemaphoreType.DMA,
        ],
    )
    def deg_kernel(e_hbm, deg_hbm, acc_v, idx_v, sem):
        c = lax.axis_index("c")
        s = lax.axis_index("s")
        w = c * NS + s
        pltpu.async_copy(e_hbm.at[1, pl.ds(w * epw, epw)], idx_v, sem)

        @plsc.parallel_loop(0, npad // 16, unroll=8)
        def _z(j):
            acc_v[pl.ds(j * 16, 16)] = jnp.zeros((16,), jnp.float32)

        pltpu.make_async_copy(e_hbm.at[1, pl.ds(0, epw)], idx_v, sem).wait()
        ones = jnp.ones((16,), jnp.float32)

        @plsc.parallel_loop(0, epw // 16, unroll=4)
        def _h(i):
            cc = idx_v[pl.ds(i * 16, 16)]
            plsc.addupdate_scatter(acc_v, [cc], ones)

        pltpu.sync_copy(acc_v, deg_hbm.at[c, s])

    return deg_kernel(eip)


def _sc_message(gt, eip, npad, nch0, nch1, k2):
    assert nch0 % 2 == 0 and nch1 % 2 == 0
    hq = H // NQ
    mesh = plsc.VectorSubcoreMesh(core_axis_name="c", subcore_axis_name="s")

    @functools.partial(
        pl.kernel, mesh=mesh,
        compiler_params=pltpu.CompilerParams(use_tc_tiling_on_sc=False,
                                             needs_layout_passes=False),
        out_type=jax.ShapeDtypeStruct((NC, NS, NQ, npad), jnp.float32),
        scratch_types=[
            pltpu.VMEM((hq, npad), jnp.float32),
            pltpu.VMEM((hq, npad), jnp.float32),
            pltpu.VMEM((2, k2), jnp.int32),
            pltpu.VMEM((2, k2), jnp.int32),
            pltpu.SemaphoreType.DMA,
            pltpu.SemaphoreType.DMA,
            pltpu.SemaphoreType.DMA,
        ],
    )
    def msg_kernel(gt_hbm, e_hbm, out_hbm,
                   gq_v, acc_v, ridx, cidx, semg, sem0, sem1):
        c = lax.axis_index("c")
        s = lax.axis_index("s")
        q = s % NQ
        p = s // NQ
        nch = lax.select(c == 0, nch0, nch1)
        base = lax.select(c == 0, p * nch0, NP * nch0 + p * nch1) * k2

        def ld(t, buf, sem):
            off = base + t * k2
            pltpu.async_copy(e_hbm.at[0, pl.ds(off, k2)], ridx.at[buf], sem)
            pltpu.async_copy(e_hbm.at[1, pl.ds(off, k2)], cidx.at[buf], sem)

        pltpu.async_copy(gt_hbm.at[pl.ds(q * hq, hq)], gq_v, semg)
        ld(0, 0, sem0)

        @plsc.parallel_loop(0, npad // 16, unroll=8)
        def _z(j):
            for f in range(hq):
                acc_v[f, pl.ds(j * 16, 16)] = jnp.zeros((16,), jnp.float32)

        pltpu.make_async_copy(gt_hbm.at[pl.ds(0, hq)], gq_v, semg).wait()

        fvecs = [jnp.full((16,), f, jnp.int32) for f in range(hq)]

        def chunk(t, buf, sem_cur, sem_nxt):
            @pl.when(t + 1 < nch)
            def _():
                ld(t + 1, 1 - buf, sem_nxt)

            rb = ridx.at[buf]
            cb = cidx.at[buf]
            pltpu.make_async_copy(e_hbm.at[0, pl.ds(0, k2)], rb,
                                  sem_cur).wait()
            pltpu.make_async_copy(e_hbm.at[0, pl.ds(0, k2)], cb,
                                  sem_cur).wait()

            @plsc.parallel_loop(0, k2 // 16, unroll=4)
            def _i(i):
                r = rb[pl.ds(i * 16, 16)]
                cc = cb[pl.ds(i * 16, 16)]
                for f in range(hq):
                    v = plsc.load_gather(gq_v, [fvecs[f], r])
                    plsc.addupdate_scatter(acc_v, [fvecs[f], cc], v)

        def body(t2, carry):
            t = t2 * 2
            chunk(t, 0, sem0, sem1)
            chunk(t + 1, 1, sem1, sem0)
            return carry

        lax.fori_loop(0, nch // 2, body, 0)
        pltpu.sync_copy(acc_v, out_hbm.at[c, s])

    return msg_kernel(gt, eip)


def _tc_linear1(x, w1, b1, degp, npad):
    n, d = x.shape

    def body(x_ref, w_ref, b_ref, deg_ref, g_ref, dinv_ref):
        deg = jnp.sum(deg_ref[...], axis=(0, 1)) + 1.0
        dinv = lax.rsqrt(deg)
        hid = lax.dot_general(w_ref[...], x_ref[...],
                              (((1,), (1,)), ((), ())),
                              preferred_element_type=jnp.float32)
        g_ref[...] = dinv[None, :] * (hid + b_ref[...][:, None])
        dinv_ref[...] = dinv

    return pl.pallas_call(
        body,
        grid=(npad // BM,),
        in_specs=[
            pl.BlockSpec((BM, d), lambda i: (i, 0)),
            pl.BlockSpec((H, d), lambda i: (0, 0)),
            pl.BlockSpec((H,), lambda i: (0,)),
            pl.BlockSpec((NC, NS, BM), lambda i: (0, 0, i)),
        ],
        out_specs=[
            pl.BlockSpec((H, BM), lambda i: (0, i)),
            pl.BlockSpec((BM,), lambda i: (i,)),
        ],
        out_shape=[
            jax.ShapeDtypeStruct((H, npad), jnp.float32),
            jax.ShapeDtypeStruct((npad,), jnp.float32),
        ],
    )(x, w1, b1, degp)


def _assemble(s_ref):
    rows = []
    for q in range(NQ):
        t = None
        for c in range(NC):
            for p in range(NP):
                term = s_ref[c, p * NQ + q]
                t = term if t is None else t + term
        rows.append(t)
    return jnp.concatenate(rows, axis=0)


def _tc_mid(s1, gt, dinv, w2p, b2p, npad):
    def body(s_ref, g_ref, dinv_ref, w_ref, b_ref, o_ref):
        di = dinv_ref[...]
        h1 = jnp.maximum(di[None, :] * (_assemble(s_ref) + g_ref[...]), 0.0)
        h2 = lax.dot_general(w_ref[...], h1, (((1,), (0,)), ((), ())),
                             preferred_element_type=jnp.float32)
        o_ref[...] = di[None, :] * (h2 + b_ref[...][:, None])

    return pl.pallas_call(
        body,
        grid=(npad // BM,),
        in_specs=[
            pl.BlockSpec((NC, NS, NQ, BM), lambda i: (0, 0, 0, i)),
            pl.BlockSpec((H, BM), lambda i: (0, i)),
            pl.BlockSpec((BM,), lambda i: (i,)),
            pl.BlockSpec((H, H), lambda i: (0, 0)),
            pl.BlockSpec((H,), lambda i: (0,)),
        ],
        out_specs=pl.BlockSpec((H, BM), lambda i: (0, i)),
        out_shape=jax.ShapeDtypeStruct((H, npad), jnp.float32),
    )(s1, gt, dinv, w2p, b2p)


def _tc_out(s2, g2t, dinv, ncls, npad):
    def body(s_ref, g_ref, dinv_ref, o_ref):
        di = dinv_ref[...]
        o = di[None, :] * (_assemble(s_ref) + g_ref[...])
        logits = o[:ncls, :]
        m = jnp.max(logits, axis=0, keepdims=True)
        lse = jnp.log(jnp.sum(jnp.exp(logits - m), axis=0, keepdims=True))
        o_ref[...] = logits - m - lse

    return pl.pallas_call(
        body,
        grid=(npad // BM,),
        in_specs=[
            pl.BlockSpec((NC, NS, NQ, BM), lambda i: (0, 0, 0, i)),
            pl.BlockSpec((H, BM), lambda i: (0, i)),
            pl.BlockSpec((BM,), lambda i: (i,)),
        ],
        out_specs=pl.BlockSpec((ncls, BM), lambda i: (0, i)),
        out_shape=jax.ShapeDtypeStruct((ncls, npad), jnp.float32),
    )(s2, g2t, dinv)


def kernel(x, edge_index, W1, b1, W2, b2):
    n, d = x.shape
    hid = W1.shape[0]
    ncls = W2.shape[0]
    assert hid == H

    npad = ((n + 2 * BM - 1) // (2 * BM)) * 2 * BM
    e = edge_index.shape[1]
    k2 = 1024
    unit = 2 * NC * NP * k2
    epad = ((e + unit - 1) // unit) * unit
    pairs = epad // (NP * 2 * k2)
    a2 = max(1, min(pairs - 1, round(0.65 * pairs)))
    nch0, nch1 = 2 * a2, 2 * (pairs - a2)
    epw = epad // NW

    ei = edge_index.astype(jnp.int32)
    eip = jnp.concatenate(
        [ei, jnp.stack([jnp.zeros((epad - e,), jnp.int32),
                        jnp.full((epad - e,), n, jnp.int32)])], axis=1)

    w2p = jnp.zeros((H, H), jnp.float32).at[:ncls].set(W2)
    b2p = jnp.zeros((H,), jnp.float32).at[:ncls].set(b2)

    degp = _sc_degree(eip, npad, epw)
    gt, dinv = _tc_linear1(x, W1, b1, degp, npad)
    s1 = _sc_message(gt, eip, npad, nch0, nch1, k2)
    g2t = _tc_mid(s1, gt, dinv, w2p, b2p, npad)
    s2 = _sc_message(g2t, eip, npad, nch0, nch1, k2)
    out_t = _tc_out(s2, g2t, dinv, ncls, npad)
    return out_t[:, :n].T


# speedup vs baseline: 37.2497x; 1.0553x over previous
import functools

import jax
import jax.numpy as jnp
from jax import lax
from jax.experimental import pallas as pl
from jax.experimental.pallas import tpu as pltpu
from jax.experimental.pallas import tpu_sc as plsc

NC = 2
NS = 16
NW = NC * NS
NQ = 4
NP = 4
K = 2048
H = 16
BM = 1024


def _sc_degree(eip, npad, epw):
    mesh = plsc.VectorSubcoreMesh(core_axis_name="c", subcore_axis_name="s")

    @functools.partial(
        pl.kernel, mesh=mesh,
        compiler_params=pltpu.CompilerParams(use_tc_tiling_on_sc=False,
                                             needs_layout_passes=False),
        out_type=jax.ShapeDtypeStruct((NC, NS, npad), jnp.float32),
        scratch_types=[
            pltpu.VMEM((npad,), jnp.float32),
            pltpu.VMEM((epw,), jnp.int32),
            pltpu.SemaphoreType.DMA,
        ],
    )
    def deg_kernel(e_hbm, deg_hbm, acc_v, idx_v, sem):
        c = lax.axis_index("c")
        s = lax.axis_index("s")
        w = c * NS + s
        pltpu.async_copy(e_hbm.at[1, pl.ds(w * epw, epw)], idx_v, sem)

        @plsc.parallel_loop(0, npad // 16, unroll=8)
        def _z(j):
            acc_v[pl.ds(j * 16, 16)] = jnp.zeros((16,), jnp.float32)

        pltpu.make_async_copy(e_hbm.at[1, pl.ds(0, epw)], idx_v, sem).wait()
        ones = jnp.ones((16,), jnp.float32)

        @plsc.parallel_loop(0, epw // 16, unroll=4)
        def _h(i):
            cc = idx_v[pl.ds(i * 16, 16)]
            plsc.addupdate_scatter(acc_v, [cc], ones)

        pltpu.sync_copy(acc_v, deg_hbm.at[c, s])

    return deg_kernel(eip)


def _sc_message(gt, eip, npad, nch0, nch1, k2):
    assert nch0 % 2 == 0 and nch1 % 2 == 0
    hq = H // NQ
    mesh = plsc.VectorSubcoreMesh(core_axis_name="c", subcore_axis_name="s")

    @functools.partial(
        pl.kernel, mesh=mesh,
        compiler_params=pltpu.CompilerParams(use_tc_tiling_on_sc=False,
                                             needs_layout_passes=False),
        out_type=jax.ShapeDtypeStruct((NC, NS, NQ, npad), jnp.float32),
        scratch_types=[
            pltpu.VMEM((hq, npad), jnp.float32),
            pltpu.VMEM((hq, npad), jnp.float32),
            pltpu.VMEM((2, k2), jnp.int32),
            pltpu.VMEM((2, k2), jnp.int32),
            pltpu.SemaphoreType.DMA,
            pltpu.SemaphoreType.DMA,
            pltpu.SemaphoreType.DMA,
        ],
    )
    def msg_kernel(gt_hbm, e_hbm, out_hbm,
                   gq_v, acc_v, ridx, cidx, semg, sem0, sem1):
        c = lax.axis_index("c")
        s = lax.axis_index("s")
        q = s % NQ
        p = s // NQ
        nch = lax.select(c == 0, nch0, nch1)
        base = lax.select(c == 0, p * nch0, NP * nch0 + p * nch1) * k2

        def ld(t, buf, sem):
            off = base + t * k2
            pltpu.async_copy(e_hbm.at[0, pl.ds(off, k2)], ridx.at[buf], sem)
            pltpu.async_copy(e_hbm.at[1, pl.ds(off, k2)], cidx.at[buf], sem)

        pltpu.async_copy(gt_hbm.at[pl.ds(q * hq, hq)], gq_v, semg)
        ld(0, 0, sem0)

        @plsc.parallel_loop(0, npad // 16, unroll=8)
        def _z(j):
            for f in range(hq):
                acc_v[f, pl.ds(j * 16, 16)] = jnp.zeros((16,), jnp.float32)

        pltpu.make_async_copy(gt_hbm.at[pl.ds(0, hq)], gq_v, semg).wait()

        fvecs = [jnp.full((16,), f, jnp.int32) for f in range(hq)]

        def chunk(t, buf, sem_cur, sem_nxt):
            @pl.when(t + 1 < nch)
            def _():
                ld(t + 1, 1 - buf, sem_nxt)

            rb = ridx.at[buf]
            cb = cidx.at[buf]
            pltpu.make_async_copy(e_hbm.at[0, pl.ds(0, k2)], rb,
                                  sem_cur).wait()
            pltpu.make_async_copy(e_hbm.at[0, pl.ds(0, k2)], cb,
                                  sem_cur).wait()

            @plsc.parallel_loop(0, k2 // 16, unroll=4)
            def _i(i):
                r = rb[pl.ds(i * 16, 16)]
                cc = cb[pl.ds(i * 16, 16)]
                for f in range(hq):
                    v = plsc.load_gather(gq_v, [fvecs[f], r])
                    plsc.addupdate_scatter(acc_v, [fvecs[f], cc], v)

        def body(t2, carry):
            t = t2 * 2
            chunk(t, 0, sem0, sem1)
            chunk(t + 1, 1, sem1, sem0)
            return carry

        lax.fori_loop(0, nch // 2, body, 0)
        pltpu.sync_copy(acc_v, out_hbm.at[c, s])

    return msg_kernel(gt, eip)


def _tc_linear1(x, w1, b1, degp, npad):
    n, d = x.shape
    bm = 2 * BM

    def body(x_ref, w_ref, b_ref, deg_ref, g_ref, dinv_ref):
        deg = jnp.sum(deg_ref[...], axis=(0, 1)) + 1.0
        dinv = lax.rsqrt(deg)
        hid = lax.dot_general(w_ref[...], x_ref[...],
                              (((1,), (1,)), ((), ())),
                              preferred_element_type=jnp.float32)
        g_ref[...] = dinv[None, :] * (hid + b_ref[...][:, None])
        dinv_ref[...] = dinv

    return pl.pallas_call(
        body,
        grid=(npad // bm,),
        in_specs=[
            pl.BlockSpec((bm, d), lambda i: (i, 0)),
            pl.BlockSpec((H, d), lambda i: (0, 0)),
            pl.BlockSpec((H,), lambda i: (0,)),
            pl.BlockSpec((NC, NS, bm), lambda i: (0, 0, i)),
        ],
        out_specs=[
            pl.BlockSpec((H, bm), lambda i: (0, i)),
            pl.BlockSpec((bm,), lambda i: (i,)),
        ],
        out_shape=[
            jax.ShapeDtypeStruct((H, npad), jnp.float32),
            jax.ShapeDtypeStruct((npad,), jnp.float32),
        ],
    )(x, w1, b1, degp)


def _assemble(s_ref):
    rows = []
    for q in range(NQ):
        t = None
        for c in range(NC):
            for p in range(NP):
                term = s_ref[c, p * NQ + q]
                t = term if t is None else t + term
        rows.append(t)
    return jnp.concatenate(rows, axis=0)


def _tc_mid(s1, gt, dinv, w2p, b2p, npad):
    def body(s_ref, g_ref, dinv_ref, w_ref, b_ref, o_ref):
        di = dinv_ref[...]
        h1 = jnp.maximum(di[None, :] * (_assemble(s_ref) + g_ref[...]), 0.0)
        h2 = lax.dot_general(w_ref[...], h1, (((1,), (0,)), ((), ())),
                             preferred_element_type=jnp.float32)
        o_ref[...] = di[None, :] * (h2 + b_ref[...][:, None])

    return pl.pallas_call(
        body,
        grid=(npad // BM,),
        in_specs=[
            pl.BlockSpec((NC, NS, NQ, BM), lambda i: (0, 0, 0, i)),
            pl.BlockSpec((H, BM), lambda i: (0, i)),
            pl.BlockSpec((BM,), lambda i: (i,)),
            pl.BlockSpec((H, H), lambda i: (0, 0)),
            pl.BlockSpec((H,), lambda i: (0,)),
        ],
        out_specs=pl.BlockSpec((H, BM), lambda i: (0, i)),
        out_shape=jax.ShapeDtypeStruct((H, npad), jnp.float32),
    )(s1, gt, dinv, w2p, b2p)


def _tc_out(s2, g2t, dinv, ncls, npad):
    def body(s_ref, g_ref, dinv_ref, o_ref):
        di = dinv_ref[...]
        o = di[None, :] * (_assemble(s_ref) + g_ref[...])
        logits = o[:ncls, :]
        m = jnp.max(logits, axis=0, keepdims=True)
        lse = jnp.log(jnp.sum(jnp.exp(logits - m), axis=0, keepdims=True))
        o_ref[...] = logits - m - lse

    return pl.pallas_call(
        body,
        grid=(npad // BM,),
        in_specs=[
            pl.BlockSpec((NC, NS, NQ, BM), lambda i: (0, 0, 0, i)),
            pl.BlockSpec((H, BM), lambda i: (0, i)),
            pl.BlockSpec((BM,), lambda i: (i,)),
        ],
        out_specs=pl.BlockSpec((ncls, BM), lambda i: (0, i)),
        out_shape=jax.ShapeDtypeStruct((ncls, npad), jnp.float32),
    )(s2, g2t, dinv)


def kernel(x, edge_index, W1, b1, W2, b2):
    n, d = x.shape
    hid = W1.shape[0]
    ncls = W2.shape[0]
    assert hid == H

    npad = ((n + 2 * BM - 1) // (2 * BM)) * 2 * BM
    e = edge_index.shape[1]
    k2 = 1024
    unit = 2 * NC * NP * k2
    epad = ((e + unit - 1) // unit) * unit
    pairs = epad // (NP * 2 * k2)
    a2 = max(1, min(pairs - 1, round(0.75 * pairs)))
    nch0, nch1 = 2 * a2, 2 * (pairs - a2)
    epw = epad // NW

    ei = edge_index.astype(jnp.int32)
    eip = jnp.concatenate(
        [ei, jnp.stack([jnp.zeros((epad - e,), jnp.int32),
                        jnp.full((epad - e,), n, jnp.int32)])], axis=1)

    w2p = jnp.zeros((H, H), jnp.float32).at[:ncls].set(W2)
    b2p = jnp.zeros((H,), jnp.float32).at[:ncls].set(b2)

    degp = _sc_degree(eip, npad, epw)
    gt, dinv = _tc_linear1(x, W1, b1, degp, npad)
    s1 = _sc_message(gt, eip, npad, nch0, nch1, k2)
    g2t = _tc_mid(s1, gt, dinv, w2p, b2p, npad)
    s2 = _sc_message(g2t, eip, npad, nch0, nch1, k2)
    out_t = _tc_out(s2, g2t, dinv, ncls, npad)
    return out_t[:, :n].T


# speedup vs baseline: 42.1290x; 1.1310x over previous
import functools

import jax
import jax.numpy as jnp
from jax import lax
from jax.experimental import pallas as pl
from jax.experimental.pallas import tpu as pltpu
from jax.experimental.pallas import tpu_sc as plsc

NC = 2
NS = 16
NW = NC * NS
NQ = 4
NP = 4
K = 2048
H = 16
BM = 1024


def _sc_degree(e1, epad, npad, epw):
    mesh = plsc.VectorSubcoreMesh(core_axis_name="c", subcore_axis_name="s")

    @functools.partial(
        pl.kernel, mesh=mesh,
        compiler_params=pltpu.CompilerParams(use_tc_tiling_on_sc=True,
                                             needs_layout_passes=False),
        out_type=jax.ShapeDtypeStruct((NC, NS, npad), jnp.float32),
        scratch_types=[
            pltpu.VMEM((npad,), jnp.float32),
            pltpu.VMEM((epw,), jnp.int32),
            pltpu.SemaphoreType.DMA,
        ],
    )
    def deg_kernel(e_hbm, deg_hbm, acc_v, idx_v, sem):
        c = lax.axis_index("c")
        s = lax.axis_index("s")
        w = c * NS + s
        pltpu.async_copy(e_hbm.at[pl.ds(epad + w * epw, epw)], idx_v, sem)

        @plsc.parallel_loop(0, npad // 16, unroll=8)
        def _z(j):
            acc_v[pl.ds(j * 16, 16)] = jnp.zeros((16,), jnp.float32)

        pltpu.make_async_copy(e_hbm.at[pl.ds(0, epw)], idx_v, sem).wait()
        ones = jnp.ones((16,), jnp.float32)

        @plsc.parallel_loop(0, epw // 16, unroll=4)
        def _h(i):
            cc = idx_v[pl.ds(i * 16, 16)]
            plsc.addupdate_scatter(acc_v, [cc], ones)

        pltpu.sync_copy(acc_v, deg_hbm.at[c, s])

    return deg_kernel(e1)


def _sc_message(gt, e1, epad, npad, nch0, nch1, k2):
    assert nch0 % 2 == 0 and nch1 % 2 == 0
    hq = H // NQ
    mesh = plsc.VectorSubcoreMesh(core_axis_name="c", subcore_axis_name="s")

    @functools.partial(
        pl.kernel, mesh=mesh,
        compiler_params=pltpu.CompilerParams(use_tc_tiling_on_sc=True,
                                             needs_layout_passes=False),
        out_type=jax.ShapeDtypeStruct((NC, NS, NQ, npad), jnp.float32),
        scratch_types=[
            pltpu.VMEM((hq, npad), jnp.float32),
            pltpu.VMEM((hq, npad), jnp.float32),
            pltpu.VMEM((k2,), jnp.int32),
            pltpu.VMEM((k2,), jnp.int32),
            pltpu.VMEM((k2,), jnp.int32),
            pltpu.VMEM((k2,), jnp.int32),
            pltpu.SemaphoreType.DMA,
            pltpu.SemaphoreType.DMA,
            pltpu.SemaphoreType.DMA,
        ],
    )
    def msg_kernel(gt_hbm, e_hbm, out_hbm,
                   gq_v, acc_v, ridx0, ridx1, cidx0, cidx1,
                   semg, sem0, sem1):
        ridx = (ridx0, ridx1)
        cidx = (cidx0, cidx1)
        c = lax.axis_index("c")
        s = lax.axis_index("s")
        q = s % NQ
        p = s // NQ
        nch = lax.select(c == 0, nch0, nch1)
        base = lax.select(c == 0, p * nch0, NP * nch0 + p * nch1) * k2

        def ld(t, buf, sem):
            off = base + t * k2
            pltpu.async_copy(e_hbm.at[pl.ds(off, k2)], ridx[buf], sem)
            pltpu.async_copy(e_hbm.at[pl.ds(epad + off, k2)], cidx[buf],
                             sem)

        pltpu.async_copy(gt_hbm.at[pl.ds(q * hq, hq)], gq_v, semg)
        ld(0, 0, sem0)

        @plsc.parallel_loop(0, npad // 16, unroll=8)
        def _z(j):
            for f in range(hq):
                acc_v[f, pl.ds(j * 16, 16)] = jnp.zeros((16,), jnp.float32)

        pltpu.make_async_copy(gt_hbm.at[pl.ds(0, hq)], gq_v, semg).wait()

        fvecs = [jnp.full((16,), f, jnp.int32) for f in range(hq)]

        def chunk(t, buf, sem_cur, sem_nxt):
            @pl.when(t + 1 < nch)
            def _():
                ld(t + 1, 1 - buf, sem_nxt)

            rb = ridx[buf]
            cb = cidx[buf]
            pltpu.make_async_copy(e_hbm.at[pl.ds(0, k2)], rb,
                                  sem_cur).wait()
            pltpu.make_async_copy(e_hbm.at[pl.ds(0, k2)], cb,
                                  sem_cur).wait()

            @plsc.parallel_loop(0, k2 // 16, unroll=4)
            def _i(i):
                r = rb[pl.ds(i * 16, 16)]
                cc = cb[pl.ds(i * 16, 16)]
                for f in range(hq):
                    v = plsc.load_gather(gq_v, [fvecs[f], r])
                    plsc.addupdate_scatter(acc_v, [fvecs[f], cc], v)

        def body(t2, carry):
            t = t2 * 2
            chunk(t, 0, sem0, sem1)
            chunk(t + 1, 1, sem1, sem0)
            return carry

        lax.fori_loop(0, nch // 2, body, 0)
        pltpu.sync_copy(acc_v, out_hbm.at[c, s])

    return msg_kernel(gt, e1)


def _tc_linear1(x, w1, b1, degp, npad):
    n, d = x.shape
    bm = 2 * BM

    def body(x_ref, w_ref, b_ref, deg_ref, g_ref, dinv_ref):
        deg = jnp.sum(deg_ref[...], axis=(0, 1)) + 1.0
        dinv = lax.rsqrt(deg)
        hid = lax.dot_general(w_ref[...], x_ref[...],
                              (((1,), (1,)), ((), ())),
                              preferred_element_type=jnp.float32)
        g_ref[...] = dinv[None, :] * (hid + b_ref[...][:, None])
        dinv_ref[...] = dinv

    return pl.pallas_call(
        body,
        grid=(npad // bm,),
        in_specs=[
            pl.BlockSpec((bm, d), lambda i: (i, 0)),
            pl.BlockSpec((H, d), lambda i: (0, 0)),
            pl.BlockSpec((H,), lambda i: (0,)),
            pl.BlockSpec((NC, NS, bm), lambda i: (0, 0, i)),
        ],
        out_specs=[
            pl.BlockSpec((H, bm), lambda i: (0, i)),
            pl.BlockSpec((bm,), lambda i: (i,)),
        ],
        out_shape=[
            jax.ShapeDtypeStruct((H, npad), jnp.float32),
            jax.ShapeDtypeStruct((npad,), jnp.float32),
        ],
    )(x, w1, b1, degp)


def _assemble(s_ref):
    rows = []
    for q in range(NQ):
        t = None
        for c in range(NC):
            for p in range(NP):
                term = s_ref[c, p * NQ + q]
                t = term if t is None else t + term
        rows.append(t)
    return jnp.concatenate(rows, axis=0)


def _tc_mid(s1, gt, dinv, w2p, b2p, npad):
    def body(s_ref, g_ref, dinv_ref, w_ref, b_ref, o_ref):
        di = dinv_ref[...]
        h1 = jnp.maximum(di[None, :] * (_assemble(s_ref) + g_ref[...]), 0.0)
        h2 = lax.dot_general(w_ref[...], h1, (((1,), (0,)), ((), ())),
                             preferred_element_type=jnp.float32)
        o_ref[...] = di[None, :] * (h2 + b_ref[...][:, None])

    return pl.pallas_call(
        body,
        grid=(npad // BM,),
        in_specs=[
            pl.BlockSpec((NC, NS, NQ, BM), lambda i: (0, 0, 0, i)),
            pl.BlockSpec((H, BM), lambda i: (0, i)),
            pl.BlockSpec((BM,), lambda i: (i,)),
            pl.BlockSpec((H, H), lambda i: (0, 0)),
            pl.BlockSpec((H,), lambda i: (0,)),
        ],
        out_specs=pl.BlockSpec((H, BM), lambda i: (0, i)),
        out_shape=jax.ShapeDtypeStruct((H, npad), jnp.float32),
    )(s1, gt, dinv, w2p, b2p)


def _tc_out(s2, g2t, dinv, ncls, npad):
    def body(s_ref, g_ref, dinv_ref, o_ref):
        di = dinv_ref[...]
        o = di[None, :] * (_assemble(s_ref) + g_ref[...])
        logits = o[:ncls, :]
        m = jnp.max(logits, axis=0, keepdims=True)
        lse = jnp.log(jnp.sum(jnp.exp(logits - m), axis=0, keepdims=True))
        o_ref[...] = logits - m - lse

    return pl.pallas_call(
        body,
        grid=(npad // BM,),
        in_specs=[
            pl.BlockSpec((NC, NS, NQ, BM), lambda i: (0, 0, 0, i)),
            pl.BlockSpec((H, BM), lambda i: (0, i)),
            pl.BlockSpec((BM,), lambda i: (i,)),
        ],
        out_specs=pl.BlockSpec((ncls, BM), lambda i: (0, i)),
        out_shape=jax.ShapeDtypeStruct((ncls, npad), jnp.float32),
    )(s2, g2t, dinv)


def kernel(x, edge_index, W1, b1, W2, b2):
    n, d = x.shape
    hid = W1.shape[0]
    ncls = W2.shape[0]
    assert hid == H

    npad = ((n + 2 * BM - 1) // (2 * BM)) * 2 * BM
    e = edge_index.shape[1]
    k2 = 1024
    unit = 2 * NC * NP * k2
    epad = ((e + unit - 1) // unit) * unit
    pairs = epad // (NP * 2 * k2)
    a2 = max(1, min(pairs - 1, round(0.75 * pairs)))
    nch0, nch1 = 2 * a2, 2 * (pairs - a2)
    epw = epad // NW

    ei = edge_index.astype(jnp.int32)
    e1 = jnp.concatenate(
        [ei[0], jnp.zeros((epad - e,), jnp.int32),
         ei[1], jnp.full((epad - e,), n, jnp.int32)])

    w2p = jnp.zeros((H, H), jnp.float32).at[:ncls].set(W2)
    b2p = jnp.zeros((H,), jnp.float32).at[:ncls].set(b2)

    degp = _sc_degree(e1, epad, npad, epw)
    gt, dinv = _tc_linear1(x, W1, b1, degp, npad)
    s1 = _sc_message(gt, e1, epad, npad, nch0, nch1, k2)
    g2t = _tc_mid(s1, gt, dinv, w2p, b2p, npad)
    s2 = _sc_message(g2t, e1, epad, npad, nch0, nch1, k2)
    out_t = _tc_out(s2, g2t, dinv, ncls, npad)
    return out_t[:, :n].T


# speedup vs baseline: 43.7073x; 1.0375x over previous
import functools

import jax
import jax.numpy as jnp
from jax import lax
from jax.experimental import pallas as pl
from jax.experimental.pallas import tpu as pltpu
from jax.experimental.pallas import tpu_sc as plsc

NC = 2
NS = 16
NW = NC * NS
NQ = 4
NP = 4
K = 2048
H = 16
BM = 1024


def _sc_degree(e1, epad, npad, epw):
    mesh = plsc.VectorSubcoreMesh(core_axis_name="c", subcore_axis_name="s")

    @functools.partial(
        pl.kernel, mesh=mesh,
        compiler_params=pltpu.CompilerParams(use_tc_tiling_on_sc=True,
                                             needs_layout_passes=False),
        out_type=jax.ShapeDtypeStruct((NS, npad), jnp.float32),
        scratch_types=[
            pltpu.VMEM((npad,), jnp.float32),
            pltpu.VMEM((epw,), jnp.int32),
            pltpu.SemaphoreType.DMA,
        ],
    )
    def deg_kernel(e_hbm, deg_hbm, acc_v, idx_v, sem):
        c = lax.axis_index("c")
        s = lax.axis_index("s")

        @pl.when(c == 0)
        def _():
            pltpu.async_copy(e_hbm.at[pl.ds(epad + s * epw, epw)], idx_v,
                             sem)

            @plsc.parallel_loop(0, npad // 16, unroll=8)
            def _z(j):
                acc_v[pl.ds(j * 16, 16)] = jnp.zeros((16,), jnp.float32)

            pltpu.make_async_copy(e_hbm.at[pl.ds(0, epw)], idx_v,
                                  sem).wait()
            ones = jnp.ones((16,), jnp.float32)

            @plsc.parallel_loop(0, epw // 16, unroll=4)
            def _h(i):
                cc = idx_v[pl.ds(i * 16, 16)]
                plsc.addupdate_scatter(acc_v, [cc], ones)

            pltpu.sync_copy(acc_v, deg_hbm.at[s])

    return deg_kernel(e1)


def _sc_message(gt, e1, epad, npad, nch0, nch1, k2):
    assert nch0 % 2 == 0 and nch1 % 2 == 0
    hq = H // NQ
    mesh = plsc.VectorSubcoreMesh(core_axis_name="c", subcore_axis_name="s")

    @functools.partial(
        pl.kernel, mesh=mesh,
        compiler_params=pltpu.CompilerParams(use_tc_tiling_on_sc=True,
                                             needs_layout_passes=False),
        out_type=jax.ShapeDtypeStruct((NC, NS, NQ, npad), jnp.float32),
        scratch_types=[
            pltpu.VMEM((hq, npad), jnp.float32),
            pltpu.VMEM((hq, npad), jnp.float32),
            pltpu.VMEM((k2,), jnp.int32),
            pltpu.VMEM((k2,), jnp.int32),
            pltpu.VMEM((k2,), jnp.int32),
            pltpu.VMEM((k2,), jnp.int32),
            pltpu.SemaphoreType.DMA,
            pltpu.SemaphoreType.DMA,
            pltpu.SemaphoreType.DMA,
        ],
    )
    def msg_kernel(gt_hbm, e_hbm, out_hbm,
                   gq_v, acc_v, ridx0, ridx1, cidx0, cidx1,
                   semg, sem0, sem1):
        ridx = (ridx0, ridx1)
        cidx = (cidx0, cidx1)
        c = lax.axis_index("c")
        s = lax.axis_index("s")
        q = s % NQ
        p = s // NQ
        nch = lax.select(c == 0, nch0, nch1)
        base = lax.select(c == 0, p * nch0, NP * nch0 + p * nch1) * k2

        def ld(t, buf, sem):
            off = base + t * k2
            pltpu.async_copy(e_hbm.at[pl.ds(off, k2)], ridx[buf], sem)
            pltpu.async_copy(e_hbm.at[pl.ds(epad + off, k2)], cidx[buf],
                             sem)

        pltpu.async_copy(gt_hbm.at[pl.ds(q * hq, hq)], gq_v, semg)
        ld(0, 0, sem0)

        @plsc.parallel_loop(0, npad // 16, unroll=8)
        def _z(j):
            for f in range(hq):
                acc_v[f, pl.ds(j * 16, 16)] = jnp.zeros((16,), jnp.float32)

        pltpu.make_async_copy(gt_hbm.at[pl.ds(0, hq)], gq_v, semg).wait()

        fvecs = [jnp.full((16,), f, jnp.int32) for f in range(hq)]

        def chunk(t, buf, sem_cur, sem_nxt):
            @pl.when(t + 1 < nch)
            def _():
                ld(t + 1, 1 - buf, sem_nxt)

            rb = ridx[buf]
            cb = cidx[buf]
            pltpu.make_async_copy(e_hbm.at[pl.ds(0, k2)], rb,
                                  sem_cur).wait()
            pltpu.make_async_copy(e_hbm.at[pl.ds(0, k2)], cb,
                                  sem_cur).wait()

            @plsc.parallel_loop(0, k2 // 16, unroll=4)
            def _i(i):
                r = rb[pl.ds(i * 16, 16)]
                cc = cb[pl.ds(i * 16, 16)]
                for f in range(hq):
                    v = plsc.load_gather(gq_v, [fvecs[f], r])
                    plsc.addupdate_scatter(acc_v, [fvecs[f], cc], v)

        def body(t2, carry):
            t = t2 * 2
            chunk(t, 0, sem0, sem1)
            chunk(t + 1, 1, sem1, sem0)
            return carry

        lax.fori_loop(0, nch // 2, body, 0)
        pltpu.sync_copy(acc_v, out_hbm.at[c, s])

    return msg_kernel(gt, e1)


def _tc_linear1(x, w1, b1, degp, npad):
    n, d = x.shape
    bm = 2 * BM

    def body(x_ref, w_ref, b_ref, deg_ref, g_ref, dinv_ref):
        deg = jnp.sum(deg_ref[...], axis=0) + 1.0
        dinv = lax.rsqrt(deg)
        hid = lax.dot_general(w_ref[...], x_ref[...],
                              (((1,), (1,)), ((), ())),
                              preferred_element_type=jnp.float32)
        g_ref[...] = dinv[None, :] * (hid + b_ref[...][:, None])
        dinv_ref[...] = dinv

    return pl.pallas_call(
        body,
        grid=(npad // bm,),
        in_specs=[
            pl.BlockSpec((bm, d), lambda i: (i, 0)),
            pl.BlockSpec((H, d), lambda i: (0, 0)),
            pl.BlockSpec((H,), lambda i: (0,)),
            pl.BlockSpec((NS, bm), lambda i: (0, i)),
        ],
        out_specs=[
            pl.BlockSpec((H, bm), lambda i: (0, i)),
            pl.BlockSpec((bm,), lambda i: (i,)),
        ],
        out_shape=[
            jax.ShapeDtypeStruct((H, npad), jnp.float32),
            jax.ShapeDtypeStruct((npad,), jnp.float32),
        ],
    )(x, w1, b1, degp)


def _assemble(s_ref):
    rows = []
    for q in range(NQ):
        t = None
        for c in range(NC):
            for p in range(NP):
                term = s_ref[c, p * NQ + q]
                t = term if t is None else t + term
        rows.append(t)
    return jnp.concatenate(rows, axis=0)


def _tc_mid(s1, gt, dinv, w2p, b2p, npad):
    def body(s_ref, g_ref, dinv_ref, w_ref, b_ref, o_ref):
        di = dinv_ref[...]
        h1 = jnp.maximum(di[None, :] * (_assemble(s_ref) + g_ref[...]), 0.0)
        h2 = lax.dot_general(w_ref[...], h1, (((1,), (0,)), ((), ())),
                             preferred_element_type=jnp.float32)
        o_ref[...] = di[None, :] * (h2 + b_ref[...][:, None])

    return pl.pallas_call(
        body,
        grid=(npad // BM,),
        in_specs=[
            pl.BlockSpec((NC, NS, NQ, BM), lambda i: (0, 0, 0, i)),
            pl.BlockSpec((H, BM), lambda i: (0, i)),
            pl.BlockSpec((BM,), lambda i: (i,)),
            pl.BlockSpec((H, H), lambda i: (0, 0)),
            pl.BlockSpec((H,), lambda i: (0,)),
        ],
        out_specs=pl.BlockSpec((H, BM), lambda i: (0, i)),
        out_shape=jax.ShapeDtypeStruct((H, npad), jnp.float32),
    )(s1, gt, dinv, w2p, b2p)


def _tc_out(s2, g2t, dinv, ncls, npad):
    def body(s_ref, g_ref, dinv_ref, o_ref):
        di = dinv_ref[...]
        o = di[None, :] * (_assemble(s_ref) + g_ref[...])
        logits = o[:ncls, :]
        m = jnp.max(logits, axis=0, keepdims=True)
        lse = jnp.log(jnp.sum(jnp.exp(logits - m), axis=0, keepdims=True))
        o_ref[...] = logits - m - lse

    return pl.pallas_call(
        body,
        grid=(npad // BM,),
        in_specs=[
            pl.BlockSpec((NC, NS, NQ, BM), lambda i: (0, 0, 0, i)),
            pl.BlockSpec((H, BM), lambda i: (0, i)),
            pl.BlockSpec((BM,), lambda i: (i,)),
        ],
        out_specs=pl.BlockSpec((ncls, BM), lambda i: (0, i)),
        out_shape=jax.ShapeDtypeStruct((ncls, npad), jnp.float32),
    )(s2, g2t, dinv)


def kernel(x, edge_index, W1, b1, W2, b2):
    n, d = x.shape
    hid = W1.shape[0]
    ncls = W2.shape[0]
    assert hid == H

    npad = ((n + 2 * BM - 1) // (2 * BM)) * 2 * BM
    e = edge_index.shape[1]
    k2 = 1024
    unit = 2 * NC * NP * k2
    epad = ((e + unit - 1) // unit) * unit
    pairs = epad // (NP * 2 * k2)
    a2 = max(1, min(pairs - 1, round(0.75 * pairs)))
    nch0, nch1 = 2 * a2, 2 * (pairs - a2)
    epw = epad // NS

    ei = edge_index.astype(jnp.int32)
    e1 = jnp.pad(ei, ((0, 0), (0, epad - e)),
                 constant_values=n).reshape(2 * epad)

    w2p = jnp.zeros((H, H), jnp.float32).at[:ncls].set(W2)
    b2p = jnp.zeros((H,), jnp.float32).at[:ncls].set(b2)

    degp = _sc_degree(e1, epad, npad, epw)
    gt, dinv = _tc_linear1(x, W1, b1, degp, npad)
    s1 = _sc_message(gt, e1, epad, npad, nch0, nch1, k2)
    g2t = _tc_mid(s1, gt, dinv, w2p, b2p, npad)
    s2 = _sc_message(g2t, e1, epad, npad, nch0, nch1, k2)
    out_t = _tc_out(s2, g2t, dinv, ncls, npad)
    return out_t[:, :n].T
